# baseline (speedup 1.0000x reference)
import jax
import jax.numpy as jnp
from jax import lax
from jax.experimental import pallas as pl
from jax.experimental.pallas import tpu as pltpu
from jax.experimental.pallas import tpu_sc as plsc

B = 4
T = 8192
D = 1024
NC = 2
NS = 16
NW = NC * NS
TPW = T // NW
CH = 8
DC = 512
ND = D // DC
NT = TPW // CH
NCH = NT * ND
NBUF = 6
OUT_SLACK = 3
IN_AHEAD = NBUF - OUT_SLACK
UNROLL = 4
_LOG_ND = ND.bit_length() - 1
_LOG_DC = DC.bit_length() - 1


def _sc_body(x_hbm, p_hbm, out_hbm, *scratch):
    xbufs = scratch[0:NBUF]
    pbufs = scratch[NBUF:2 * NBUF]
    sxs = scratch[2 * NBUF:3 * NBUF]
    sps = scratch[3 * NBUF:4 * NBUF]
    sos = scratch[4 * NBUF:5 * NBUF]

    wid = lax.axis_index("s") * NC + lax.axis_index("c")
    tok0 = wid * TPW

    def offs(j):
        t0 = pl.multiple_of(tok0 + (j >> _LOG_ND) * CH, CH)
        d0 = (j & (ND - 1)) << _LOG_DC
        if not isinstance(d0, int):
            d0 = pl.multiple_of(d0, DC)
        return t0, d0

    def start_in(j, r):
        t0, d0 = offs(j)
        pltpu.async_copy(p_hbm.at[pl.ds(t0, CH), pl.ds(d0, DC)], pbufs[r], sps[r])
        for b in range(B):
            pltpu.async_copy(
                x_hbm.at[b, pl.ds(t0, CH), pl.ds(d0, DC)], xbufs[r].at[b], sxs[r])

    def wait_in(r):
        pltpu.make_async_copy(
            p_hbm.at[pl.ds(0, CH), pl.ds(0, DC)], pbufs[r], sps[r]).wait()
        for b in range(B):
            pltpu.make_async_copy(
                x_hbm.at[b, pl.ds(0, CH), pl.ds(0, DC)], xbufs[r].at[b], sxs[r]).wait()

    def start_out(j, r):
        t0, d0 = offs(j)
        for b in range(B):
            pltpu.async_copy(
                xbufs[r].at[b], out_hbm.at[b, pl.ds(t0, CH), pl.ds(d0, DC)], sos[r])

    def wait_out(r):
        for b in range(B):
            pltpu.make_async_copy(
                xbufs[r].at[b], out_hbm.at[b, pl.ds(0, CH), pl.ds(0, DC)], sos[r]).wait()

    def compute(r):
        xb, pb = xbufs[r], pbufs[r]

        @plsc.parallel_loop(0, CH * DC, 16, unroll=UNROLL)
        def _(o):
            c = o >> _LOG_DC
            dd = pl.multiple_of(o & (DC - 1), 16)
            pv = pb[c, pl.ds(dd, 16)]
            for b in range(B):
                plsc.addupdate(xb.at[b, c, pl.ds(dd, 16)], pv)

    for j in range(IN_AHEAD):
        start_in(j, j)

    def position(j, k):
        wait_in(k)
        compute(k)
        start_out(j, k)
        rn = (k + NBUF - OUT_SLACK) % NBUF

        @pl.when(j >= OUT_SLACK)
        def _():
            wait_out(rn)

        @pl.when(j + IN_AHEAD < NCH)
        def _():
            start_in(j + IN_AHEAD, rn)

    def group(g, carry):
        for k in range(NBUF):
            position(g * NBUF + k, k)
        return carry

    lax.fori_loop(0, NCH // NBUF, group, 0)

    for j in range((NCH // NBUF) * NBUF, NCH):
        position(j, j % NBUF)

    for j in range(max(NCH - OUT_SLACK, 0), NCH):
        wait_out(j % NBUF)


def _sc_add(x, p):
    mesh = plsc.VectorSubcoreMesh(core_axis_name="c", subcore_axis_name="s")
    k = pl.kernel(
        _sc_body,
        out_type=jax.ShapeDtypeStruct((B, T, D), jnp.float32),
        mesh=mesh,
        compiler_params=pltpu.CompilerParams(use_tc_tiling_on_sc=True),
        scratch_types=(
            [pltpu.VMEM((B, CH, DC), jnp.float32) for _ in range(NBUF)]
            + [pltpu.VMEM((CH, DC), jnp.float32) for _ in range(NBUF)]
            + [pltpu.SemaphoreType.DMA for _ in range(3 * NBUF)]
        ),
    )
    return k(x, p)


def kernel(encoded_tokens, pos_table):
    return _sc_add(encoded_tokens, pos_table)


# speedup vs baseline: 1.0190x; 1.0190x over previous
import jax
import jax.numpy as jnp
from jax import lax
from jax.experimental import pallas as pl
from jax.experimental.pallas import tpu as pltpu
from jax.experimental.pallas import tpu_sc as plsc

B = 4
T = 8192
D = 1024
NC = 2
NS = 16
NW = NC * NS
TPW = T // NW
CH = 8
DC = 512
ND = D // DC
NT = TPW // CH
NCH = NT * ND
NBUF = 6
OUT_SLACK = 2
IN_AHEAD = NBUF - OUT_SLACK
UNROLL = 4
_LOG_ND = ND.bit_length() - 1
_LOG_DC = DC.bit_length() - 1


def _sc_body(x_hbm, p_hbm, out_hbm, *scratch):
    xbufs = scratch[0:NBUF]
    pbufs = scratch[NBUF:2 * NBUF]
    sxs = scratch[2 * NBUF:3 * NBUF]
    sps = scratch[3 * NBUF:4 * NBUF]
    sos = scratch[4 * NBUF:5 * NBUF]

    wid = lax.axis_index("s") * NC + lax.axis_index("c")
    tok0 = wid * TPW

    def offs(j):
        t0 = pl.multiple_of(tok0 + (j >> _LOG_ND) * CH, CH)
        d0 = (j & (ND - 1)) << _LOG_DC
        if not isinstance(d0, int):
            d0 = pl.multiple_of(d0, DC)
        return t0, d0

    def start_in(j, r):
        t0, d0 = offs(j)
        pltpu.async_copy(p_hbm.at[pl.ds(t0, CH), pl.ds(d0, DC)], pbufs[r], sps[r])
        pltpu.async_copy(x_hbm.at[:, pl.ds(t0, CH), pl.ds(d0, DC)], xbufs[r], sxs[r])

    def wait_in(r):
        pltpu.make_async_copy(
            p_hbm.at[pl.ds(0, CH), pl.ds(0, DC)], pbufs[r], sps[r]).wait()
        pltpu.make_async_copy(
            x_hbm.at[:, pl.ds(0, CH), pl.ds(0, DC)], xbufs[r], sxs[r]).wait()

    def start_out(j, r):
        t0, d0 = offs(j)
        pltpu.async_copy(xbufs[r], out_hbm.at[:, pl.ds(t0, CH), pl.ds(d0, DC)], sos[r])

    def wait_out(r):
        pltpu.make_async_copy(
            xbufs[r], out_hbm.at[:, pl.ds(0, CH), pl.ds(0, DC)], sos[r]).wait()

    def compute(r):
        xb, pb = xbufs[r], pbufs[r]

        @plsc.parallel_loop(0, CH * DC, 16, unroll=UNROLL)
        def _(o):
            c = o >> _LOG_DC
            dd = pl.multiple_of(o & (DC - 1), 16)
            pv = pb[c, pl.ds(dd, 16)]
            for b in range(B):
                plsc.addupdate(xb.at[b, c, pl.ds(dd, 16)], pv)

    for j in range(IN_AHEAD):
        start_in(j, j)

    def position(j, k):
        wait_in(k)
        compute(k)
        start_out(j, k)
        rn = (k + NBUF - OUT_SLACK) % NBUF

        @pl.when(j >= OUT_SLACK)
        def _():
            wait_out(rn)

        @pl.when(j + IN_AHEAD < NCH)
        def _():
            start_in(j + IN_AHEAD, rn)

    def group(g, carry):
        for k in range(NBUF):
            position(g * NBUF + k, k)
        return carry

    lax.fori_loop(0, NCH // NBUF, group, 0)

    for j in range((NCH // NBUF) * NBUF, NCH):
        position(j, j % NBUF)

    for j in range(max(NCH - OUT_SLACK, 0), NCH):
        wait_out(j % NBUF)


def _sc_add(x, p):
    mesh = plsc.VectorSubcoreMesh(core_axis_name="c", subcore_axis_name="s")
    k = pl.kernel(
        _sc_body,
        out_type=jax.ShapeDtypeStruct((B, T, D), jnp.float32),
        mesh=mesh,
        compiler_params=pltpu.CompilerParams(use_tc_tiling_on_sc=True),
        scratch_types=(
            [pltpu.VMEM((B, CH, DC), jnp.float32) for _ in range(NBUF)]
            + [pltpu.VMEM((CH, DC), jnp.float32) for _ in range(NBUF)]
            + [pltpu.SemaphoreType.DMA for _ in range(3 * NBUF)]
        ),
    )
    return k(x, p)


def kernel(encoded_tokens, pos_table):
    return _sc_add(encoded_tokens, pos_table)
